# baseline (speedup 1.0000x reference)
import jax
import jax.numpy as jnp
from jax.experimental import pallas as pl
from jax.experimental.pallas import tpu as pltpu

_F = 2048
_G = 32
_D = 64
_PACK = 4
_S = _G // _PACK
_SW = _PACK * _D
_NS_ITERS = 18


def _stats_kernel(x_ref, sum_ref, gram_ref):
    k = pl.program_id(1)

    @pl.when(k == 0)
    def _init():
        sum_ref[...] = jnp.zeros_like(sum_ref)
        gram_ref[...] = jnp.zeros_like(gram_ref)

    xb = x_ref[...]
    sum_ref[...] += jnp.sum(xb, axis=0, keepdims=True)[None]
    gram_ref[0] += jax.lax.dot_general(
        xb, xb, (((0,), (0,)), ((), ())),
        preferred_element_type=jnp.float32)


def _whiten_kernel(gram_ref, mu_ref, muT_ref, n_ref, w_ref):
    n = n_ref[0]
    mu = mu_ref[0]
    muT = muT_ref[0]
    ri = jax.lax.broadcasted_iota(jnp.int32, (_SW, _SW), 0)
    ci = jax.lax.broadcasted_iota(jnp.int32, (_SW, _SW), 1)
    mask = ((ri // _D) == (ci // _D)).astype(jnp.float32)
    eye = (ri == ci).astype(jnp.float32)

    cov = (gram_ref[0] - n * (muT * mu)) * mask * (1.0 / _G)
    rowsum = jnp.sum(jnp.abs(cov), axis=-1, keepdims=True)
    s = jnp.maximum(jnp.max(rowsum), 1e-30)
    a = cov * (1.0 / s)

    y = a
    z = eye
    for _ in range(_NS_ITERS):
        zy = jax.lax.dot_general(z, y, (((1,), (0,)), ((), ())),
                                 preferred_element_type=jnp.float32)
        t = 1.5 * eye - 0.5 * zy
        y = jax.lax.dot_general(y, t, (((1,), (0,)), ((), ())),
                                preferred_element_type=jnp.float32)
        z = jax.lax.dot_general(t, z, (((1,), (0,)), ((), ())),
                                preferred_element_type=jnp.float32)
    w_ref[0] = z * jax.lax.rsqrt(s)


def _apply_kernel(x_ref, m_ref, mu_ref, o_ref):
    xc = x_ref[...] - mu_ref[...]
    o_ref[...] = jax.lax.dot_general(
        xc, m_ref[...], (((1,), (0,)), ((), ())),
        preferred_element_type=jnp.float32)


def kernel(x, shuffle_idx):
    n_rows, f = x.shape
    assert f == _F
    cf = shuffle_idx.astype(jnp.int32)

    blk = 1024
    blk_stats = 512
    p_stats = 2
    k_stats = n_rows // (blk_stats * p_stats)

    sums, gram_p = pl.pallas_call(
        _stats_kernel,
        grid=(p_stats, k_stats),
        in_specs=[pl.BlockSpec((blk_stats, _F),
                               lambda p, k: (p * k_stats + k, 0))],
        out_specs=[
            pl.BlockSpec((1, 1, _F), lambda p, k: (p, 0, 0)),
            pl.BlockSpec((1, _F, _F), lambda p, k: (p, 0, 0)),
        ],
        out_shape=[
            jax.ShapeDtypeStruct((p_stats, 1, _F), jnp.float32),
            jax.ShapeDtypeStruct((p_stats, _F, _F), jnp.float32),
        ],
        compiler_params=pltpu.CompilerParams(
            dimension_semantics=("parallel", "arbitrary")),
    )(x)

    colsum = jnp.sum(sums, axis=0)
    gram = jnp.sum(gram_p, axis=0)
    mean = colsum / n_rows

    cfs = cf.reshape(_S, _SW)
    gram_s = gram[cfs[:, :, None], cfs[:, None, :]]
    mu_flat = mean[0][cf]
    mu_s = mu_flat.reshape(_S, 1, _SW)
    muT_s = mu_flat.reshape(_S, _SW, 1)
    n_arr = jnp.full((1,), float(n_rows), jnp.float32)

    w = pl.pallas_call(
        _whiten_kernel,
        grid=(_S,),
        in_specs=[
            pl.BlockSpec((1, _SW, _SW), lambda s: (s, 0, 0)),
            pl.BlockSpec((1, 1, _SW), lambda s: (s, 0, 0)),
            pl.BlockSpec((1, _SW, 1), lambda s: (s, 0, 0)),
            pl.BlockSpec(memory_space=pltpu.SMEM),
        ],
        out_specs=pl.BlockSpec((1, _SW, _SW), lambda s: (s, 0, 0)),
        out_shape=jax.ShapeDtypeStruct((_S, _SW, _SW), jnp.float32),
        compiler_params=pltpu.CompilerParams(
            dimension_semantics=("parallel",)),
    )(gram_s, mu_s, muT_s, n_arr)

    m = jnp.zeros((_F, _F), jnp.float32)
    m = m.at[cfs[:, :, None], cfs[:, None, :]].set(w)

    p_apply = 8
    k_apply = n_rows // (blk * p_apply)
    out = pl.pallas_call(
        _apply_kernel,
        grid=(p_apply, k_apply),
        in_specs=[
            pl.BlockSpec((blk, _F), lambda p, k: (p * k_apply + k, 0)),
            pl.BlockSpec((_F, _F), lambda p, k: (0, 0)),
            pl.BlockSpec((1, _F), lambda p, k: (0, 0)),
        ],
        out_specs=pl.BlockSpec((blk, _F), lambda p, k: (p * k_apply + k, 0)),
        out_shape=jax.ShapeDtypeStruct((n_rows, _F), jnp.float32),
        compiler_params=pltpu.CompilerParams(
            dimension_semantics=("parallel", "arbitrary")),
    )(x, m, mean)

    return out


# speedup vs baseline: 3.2092x; 3.2092x over previous
import jax
import jax.numpy as jnp
from jax.experimental import pallas as pl
from jax.experimental.pallas import tpu as pltpu

_F = 2048
_G = 32
_D = 64
_PACK = 4
_S = _G // _PACK
_SW = _PACK * _D
_NS_ITERS = 18


def _stats_kernel(x_ref, sum_ref, gram_ref):
    k = pl.program_id(1)

    @pl.when(k == 0)
    def _init():
        sum_ref[...] = jnp.zeros_like(sum_ref)
        gram_ref[...] = jnp.zeros_like(gram_ref)

    xb = x_ref[...]
    sum_ref[...] += jnp.sum(xb, axis=0, keepdims=True)[None]
    gram_ref[0] += jax.lax.dot_general(
        xb, xb, (((0,), (0,)), ((), ())),
        preferred_element_type=jnp.float32)


def _whiten_kernel(gram_ref, mu_ref, muT_ref, n_ref, w_ref):
    n = n_ref[0]
    mu = mu_ref[0]
    muT = muT_ref[0]
    ri = jax.lax.broadcasted_iota(jnp.int32, (_SW, _SW), 0)
    ci = jax.lax.broadcasted_iota(jnp.int32, (_SW, _SW), 1)
    mask = ((ri // _D) == (ci // _D)).astype(jnp.float32)
    eye = (ri == ci).astype(jnp.float32)

    cov = (gram_ref[0] - n * (muT * mu)) * mask * (1.0 / _G)
    rowsum = jnp.sum(jnp.abs(cov), axis=-1, keepdims=True)
    s = jnp.maximum(jnp.max(rowsum), 1e-30)
    a = cov * (1.0 / s)

    y = a
    z = eye
    for _ in range(_NS_ITERS):
        zy = jax.lax.dot_general(z, y, (((1,), (0,)), ((), ())),
                                 preferred_element_type=jnp.float32)
        t = 1.5 * eye - 0.5 * zy
        y = jax.lax.dot_general(y, t, (((1,), (0,)), ((), ())),
                                preferred_element_type=jnp.float32)
        z = jax.lax.dot_general(t, z, (((1,), (0,)), ((), ())),
                                preferred_element_type=jnp.float32)
    w_ref[0] = z * jax.lax.rsqrt(s)


def _apply_kernel(x_ref, m_ref, mu_ref, o_ref):
    xc = x_ref[...] - mu_ref[...]
    o_ref[...] = jax.lax.dot_general(
        xc, m_ref[...], (((1,), (0,)), ((), ())),
        preferred_element_type=jnp.float32)


def kernel(x, shuffle_idx):
    n_rows, f = x.shape
    assert f == _F
    cf = shuffle_idx.astype(jnp.int32)

    blk = 1024
    blk_stats = 512
    p_stats = 2
    k_stats = n_rows // (blk_stats * p_stats)

    sums, gram_p = pl.pallas_call(
        _stats_kernel,
        grid=(p_stats, k_stats),
        in_specs=[pl.BlockSpec((blk_stats, _F),
                               lambda p, k: (p * k_stats + k, 0))],
        out_specs=[
            pl.BlockSpec((1, 1, _F), lambda p, k: (p, 0, 0)),
            pl.BlockSpec((1, _F, _F), lambda p, k: (p, 0, 0)),
        ],
        out_shape=[
            jax.ShapeDtypeStruct((p_stats, 1, _F), jnp.float32),
            jax.ShapeDtypeStruct((p_stats, _F, _F), jnp.float32),
        ],
        compiler_params=pltpu.CompilerParams(
            dimension_semantics=("parallel", "arbitrary")),
    )(x)

    colsum = jnp.sum(sums, axis=0)
    gram = jnp.sum(gram_p, axis=0)
    mean = colsum / n_rows

    cfs = cf.reshape(_S, _SW)
    gram_s = gram[cfs[:, :, None], cfs[:, None, :]]
    mu_flat = mean[0][cf]
    mu_s = mu_flat.reshape(_S, 1, _SW)
    muT_s = mu_flat.reshape(_S, _SW, 1)
    n_arr = jnp.full((1,), float(n_rows), jnp.float32)

    w = pl.pallas_call(
        _whiten_kernel,
        grid=(_S,),
        in_specs=[
            pl.BlockSpec((1, _SW, _SW), lambda s: (s, 0, 0)),
            pl.BlockSpec((1, 1, _SW), lambda s: (s, 0, 0)),
            pl.BlockSpec((1, _SW, 1), lambda s: (s, 0, 0)),
            pl.BlockSpec(memory_space=pltpu.SMEM),
        ],
        out_specs=pl.BlockSpec((1, _SW, _SW), lambda s: (s, 0, 0)),
        out_shape=jax.ShapeDtypeStruct((_S, _SW, _SW), jnp.float32),
        compiler_params=pltpu.CompilerParams(
            dimension_semantics=("parallel",)),
    )(gram_s, mu_s, muT_s, n_arr)

    wbig = jnp.zeros((_F, _F), jnp.float32)
    for s in range(_S):
        wbig = jax.lax.dynamic_update_slice(wbig, w[s], (s * _SW, s * _SW))
    inv = jnp.argsort(cf)
    m = wbig[inv][:, inv]

    p_apply = 8
    k_apply = n_rows // (blk * p_apply)
    out = pl.pallas_call(
        _apply_kernel,
        grid=(p_apply, k_apply),
        in_specs=[
            pl.BlockSpec((blk, _F), lambda p, k: (p * k_apply + k, 0)),
            pl.BlockSpec((_F, _F), lambda p, k: (0, 0)),
            pl.BlockSpec((1, _F), lambda p, k: (0, 0)),
        ],
        out_specs=pl.BlockSpec((blk, _F), lambda p, k: (p * k_apply + k, 0)),
        out_shape=jax.ShapeDtypeStruct((n_rows, _F), jnp.float32),
        compiler_params=pltpu.CompilerParams(
            dimension_semantics=("parallel", "arbitrary")),
    )(x, m, mean)

    return out


# speedup vs baseline: 3.4082x; 1.0620x over previous
import jax
import jax.numpy as jnp
from jax.experimental import pallas as pl
from jax.experimental.pallas import tpu as pltpu

_F = 2048
_G = 32
_D = 64
_PACK = 4
_S = _G // _PACK
_SW = _PACK * _D
_NS_ITERS = 18


def _stats_kernel(x_ref, sum_ref, gram_ref):
    k = pl.program_id(1)

    @pl.when(k == 0)
    def _init():
        sum_ref[...] = jnp.zeros_like(sum_ref)
        gram_ref[...] = jnp.zeros_like(gram_ref)

    xb = x_ref[...]
    sum_ref[...] += jnp.sum(xb, axis=0, keepdims=True)[None]
    gram_ref[0] += jax.lax.dot_general(
        xb, xb, (((0,), (0,)), ((), ())),
        preferred_element_type=jnp.float32)


def _dot(a, b, dims):
    return jax.lax.dot_general(a, b, (dims, ((), ())),
                               preferred_element_type=jnp.float32)


def _whiten_kernel(g_ref, cf_ref, mu_ref, n_ref, m_ref):
    k = pl.program_id(1)
    n = n_ref[0]

    ri = jax.lax.broadcasted_iota(jnp.int32, (_F, _SW), 0)
    p_s = (ri == cf_ref[0]).astype(jnp.float32)

    gp = _dot(g_ref[...], p_s, ((1,), (0,)))
    gs = _dot(p_s, gp, ((0,), (0,)))
    mu_s = _dot(mu_ref[...], p_s, ((1,), (0,)))
    outer = _dot(mu_s, mu_s, ((0,), (0,)))

    ri2 = jax.lax.broadcasted_iota(jnp.int32, (_SW, _SW), 0)
    ci2 = jax.lax.broadcasted_iota(jnp.int32, (_SW, _SW), 1)
    mask = ((ri2 // _D) == (ci2 // _D)).astype(jnp.float32)
    eye = (ri2 == ci2).astype(jnp.float32)

    cov = (gs - n * outer) * mask * (1.0 / _G)
    rowsum = jnp.sum(jnp.abs(cov), axis=-1, keepdims=True)
    s = jnp.maximum(jnp.max(rowsum), 1e-30)
    a = cov * (1.0 / s)

    y = a
    z = eye
    for _ in range(_NS_ITERS):
        zy = _dot(z, y, ((1,), (0,)))
        t = 1.5 * eye - 0.5 * zy
        y = _dot(y, t, ((1,), (0,)))
        z = _dot(t, z, ((1,), (0,)))
    w_s = z * jax.lax.rsqrt(s)

    pw = _dot(p_s, w_s, ((1,), (0,)))

    @pl.when(k == 0)
    def _init():
        m_ref[...] = jnp.zeros_like(m_ref)

    q = _F // 4
    for j in range(4):
        mcq = _dot(pw, p_s[j * q:(j + 1) * q, :], ((1,), (1,)))
        m_ref[0, :, j * q:(j + 1) * q] += mcq


def _apply_kernel(x_ref, m_ref, mu_ref, o_ref):
    xc = x_ref[...] - mu_ref[...]
    o_ref[...] = jax.lax.dot_general(
        xc, m_ref[...], (((1,), (0,)), ((), ())),
        preferred_element_type=jnp.float32)


def kernel(x, shuffle_idx):
    n_rows, f = x.shape
    assert f == _F
    cf = shuffle_idx.astype(jnp.int32)
    cf3 = cf.reshape(_S, 1, _SW)

    blk = 1024
    blk_stats = 512
    p_stats = 2
    k_stats = n_rows // (blk_stats * p_stats)

    sums, gram_p = pl.pallas_call(
        _stats_kernel,
        grid=(p_stats, k_stats),
        in_specs=[pl.BlockSpec((blk_stats, _F),
                               lambda p, k: (p * k_stats + k, 0))],
        out_specs=[
            pl.BlockSpec((1, 1, _F), lambda p, k: (p, 0, 0)),
            pl.BlockSpec((1, _F, _F), lambda p, k: (p, 0, 0)),
        ],
        out_shape=[
            jax.ShapeDtypeStruct((p_stats, 1, _F), jnp.float32),
            jax.ShapeDtypeStruct((p_stats, _F, _F), jnp.float32),
        ],
        compiler_params=pltpu.CompilerParams(
            dimension_semantics=("parallel", "arbitrary")),
    )(x)

    colsum = jnp.sum(sums, axis=0)
    g = jnp.sum(gram_p, axis=0)
    mean = colsum / n_rows
    n_arr = jnp.full((1,), float(n_rows), jnp.float32)

    p_whiten = 2
    k_whiten = _S // p_whiten
    m_p = pl.pallas_call(
        _whiten_kernel,
        grid=(p_whiten, k_whiten),
        in_specs=[
            pl.BlockSpec((_F, _F), lambda p, k: (0, 0)),
            pl.BlockSpec((1, 1, _SW), lambda p, k: (p * k_whiten + k, 0, 0)),
            pl.BlockSpec((1, _F), lambda p, k: (0, 0)),
            pl.BlockSpec(memory_space=pltpu.SMEM),
        ],
        out_specs=pl.BlockSpec((1, _F, _F), lambda p, k: (p, 0, 0)),
        out_shape=jax.ShapeDtypeStruct((p_whiten, _F, _F), jnp.float32),
        compiler_params=pltpu.CompilerParams(
            dimension_semantics=("parallel", "arbitrary")),
    )(g, cf3, mean, n_arr)

    m = jnp.sum(m_p, axis=0)

    p_apply = 8
    k_apply = n_rows // (blk * p_apply)
    out = pl.pallas_call(
        _apply_kernel,
        grid=(p_apply, k_apply),
        in_specs=[
            pl.BlockSpec((blk, _F), lambda p, k: (p * k_apply + k, 0)),
            pl.BlockSpec((_F, _F), lambda p, k: (0, 0)),
            pl.BlockSpec((1, _F), lambda p, k: (0, 0)),
        ],
        out_specs=pl.BlockSpec((blk, _F), lambda p, k: (p * k_apply + k, 0)),
        out_shape=jax.ShapeDtypeStruct((n_rows, _F), jnp.float32),
        compiler_params=pltpu.CompilerParams(
            dimension_semantics=("parallel", "arbitrary")),
    )(x, m, mean)

    return out


# speedup vs baseline: 3.6610x; 1.0742x over previous
import jax
import jax.numpy as jnp
from jax.experimental import pallas as pl
from jax.experimental.pallas import tpu as pltpu

_F = 2048
_G = 32
_D = 64
_PACK = 4
_S = _G // _PACK
_SW = _PACK * _D
_NS_ITERS = 18


def _stats_kernel(x_ref, sum_ref, gram_ref):
    k = pl.program_id(0)

    @pl.when(k == 0)
    def _init():
        sum_ref[...] = jnp.zeros_like(sum_ref)
        gram_ref[...] = jnp.zeros_like(gram_ref)

    xb = x_ref[...]
    sum_ref[...] += jnp.sum(xb, axis=0, keepdims=True)
    gram_ref[...] += jax.lax.dot_general(
        xb, xb, (((0,), (0,)), ((), ())),
        preferred_element_type=jnp.float32)


def _dot(a, b, dims):
    return jax.lax.dot_general(a, b, (dims, ((), ())),
                               preferred_element_type=jnp.float32)


def _whiten_kernel(g_ref, cf_ref, mu_ref, n_ref, m_ref):
    k = pl.program_id(0)
    n = n_ref[0]

    ri = jax.lax.broadcasted_iota(jnp.int32, (_F, _SW), 0)
    p_s = (ri == cf_ref[0]).astype(jnp.float32)

    gp = _dot(g_ref[...], p_s, ((1,), (0,)))
    gs = _dot(p_s, gp, ((0,), (0,)))
    mu_s = _dot(mu_ref[...], p_s, ((1,), (0,)))
    outer = _dot(mu_s, mu_s, ((0,), (0,)))

    ri2 = jax.lax.broadcasted_iota(jnp.int32, (_SW, _SW), 0)
    ci2 = jax.lax.broadcasted_iota(jnp.int32, (_SW, _SW), 1)
    mask = ((ri2 // _D) == (ci2 // _D)).astype(jnp.float32)
    eye = (ri2 == ci2).astype(jnp.float32)

    cov = (gs - n * outer) * mask * (1.0 / _G)
    rowsum = jnp.sum(jnp.abs(cov), axis=-1, keepdims=True)
    s = jnp.maximum(jnp.max(rowsum), 1e-30)
    a = cov * (1.0 / s)

    y = a
    z = eye
    for _ in range(_NS_ITERS):
        zy = _dot(z, y, ((1,), (0,)))
        t = 1.5 * eye - 0.5 * zy
        y = _dot(y, t, ((1,), (0,)))
        z = _dot(t, z, ((1,), (0,)))
    w_s = z * jax.lax.rsqrt(s)

    pw = _dot(p_s, w_s, ((1,), (0,)))

    @pl.when(k == 0)
    def _init():
        m_ref[...] = jnp.zeros_like(m_ref)

    q = _F // 4
    for j in range(4):
        mcq = _dot(pw, p_s[j * q:(j + 1) * q, :], ((1,), (1,)))
        m_ref[:, j * q:(j + 1) * q] += mcq


def _apply_kernel(x_ref, m_ref, mu_ref, o_ref):
    xc = x_ref[...] - mu_ref[...]
    o_ref[...] = jax.lax.dot_general(
        xc, m_ref[...], (((1,), (0,)), ((), ())),
        preferred_element_type=jnp.float32)


def kernel(x, shuffle_idx):
    n_rows, f = x.shape
    assert f == _F
    cf = shuffle_idx.astype(jnp.int32)
    cf3 = cf.reshape(_S, 1, _SW)

    blk = 1024

    colsum, gram = pl.pallas_call(
        _stats_kernel,
        grid=(n_rows // blk,),
        in_specs=[pl.BlockSpec((blk, _F), lambda k: (k, 0))],
        out_specs=[
            pl.BlockSpec((1, _F), lambda k: (0, 0)),
            pl.BlockSpec((_F, _F), lambda k: (0, 0)),
        ],
        out_shape=[
            jax.ShapeDtypeStruct((1, _F), jnp.float32),
            jax.ShapeDtypeStruct((_F, _F), jnp.float32),
        ],
        compiler_params=pltpu.CompilerParams(
            dimension_semantics=("arbitrary",)),
    )(x)

    mean = colsum / n_rows
    n_arr = jnp.full((1,), float(n_rows), jnp.float32)

    m = pl.pallas_call(
        _whiten_kernel,
        grid=(_S,),
        in_specs=[
            pl.BlockSpec((_F, _F), lambda k: (0, 0)),
            pl.BlockSpec((1, 1, _SW), lambda k: (k, 0, 0)),
            pl.BlockSpec((1, _F), lambda k: (0, 0)),
            pl.BlockSpec(memory_space=pltpu.SMEM),
        ],
        out_specs=pl.BlockSpec((_F, _F), lambda k: (0, 0)),
        out_shape=jax.ShapeDtypeStruct((_F, _F), jnp.float32),
        compiler_params=pltpu.CompilerParams(
            dimension_semantics=("arbitrary",)),
    )(gram, cf3, mean, n_arr)

    out = pl.pallas_call(
        _apply_kernel,
        grid=(n_rows // blk,),
        in_specs=[
            pl.BlockSpec((blk, _F), lambda k: (k, 0)),
            pl.BlockSpec((_F, _F), lambda k: (0, 0)),
            pl.BlockSpec((1, _F), lambda k: (0, 0)),
        ],
        out_specs=pl.BlockSpec((blk, _F), lambda k: (k, 0)),
        out_shape=jax.ShapeDtypeStruct((n_rows, _F), jnp.float32),
        compiler_params=pltpu.CompilerParams(
            dimension_semantics=("arbitrary",)),
    )(x, m, mean)

    return out


# speedup vs baseline: 3.7498x; 1.0242x over previous
import functools

import jax
import jax.numpy as jnp
from jax.experimental import pallas as pl
from jax.experimental.pallas import tpu as pltpu

_F = 2048
_G = 32
_D = 64
_PACK = 4
_S = _G // _PACK
_SW = _PACK * _D
_NS_ITERS = 12


def _stats_kernel(x_ref, mean_ref, gram_ref, *, inv_n, last_k):
    k = pl.program_id(0)

    @pl.when(k == 0)
    def _init():
        mean_ref[...] = jnp.zeros_like(mean_ref)
        gram_ref[...] = jnp.zeros_like(gram_ref)

    xb = x_ref[...]
    mean_ref[...] += jnp.sum(xb, axis=0, keepdims=True)
    gram_ref[...] += jax.lax.dot_general(
        xb, xb, (((0,), (0,)), ((), ())),
        preferred_element_type=jnp.float32)

    @pl.when(k == last_k)
    def _finish():
        mean_ref[...] *= inv_n


def _dot(a, b, dims):
    return jax.lax.dot_general(a, b, (dims, ((), ())),
                               preferred_element_type=jnp.float32)


def _whiten_kernel(g_ref, cf_ref, mu_ref, m_ref, *, n):
    k = pl.program_id(0)

    ri = jax.lax.broadcasted_iota(jnp.int32, (_F, _SW), 0)
    p_s = (ri == cf_ref[0]).astype(jnp.float32)

    gp = _dot(g_ref[...], p_s, ((1,), (0,)))
    gs = _dot(p_s, gp, ((0,), (0,)))
    mu_s = _dot(mu_ref[...], p_s, ((1,), (0,)))
    outer = _dot(mu_s, mu_s, ((0,), (0,)))

    ri2 = jax.lax.broadcasted_iota(jnp.int32, (_SW, _SW), 0)
    ci2 = jax.lax.broadcasted_iota(jnp.int32, (_SW, _SW), 1)
    mask = ((ri2 // _D) == (ci2 // _D)).astype(jnp.float32)
    eye = (ri2 == ci2).astype(jnp.float32)

    cov = (gs - n * outer) * mask * (1.0 / _G)
    rowsum = jnp.sum(jnp.abs(cov), axis=-1, keepdims=True)
    s = jnp.maximum(jnp.max(rowsum), 1e-30)
    a = cov * (1.0 / s)

    y = a
    z = eye
    for _ in range(_NS_ITERS):
        zy = _dot(z, y, ((1,), (0,)))
        t = 1.5 * eye - 0.5 * zy
        y = _dot(y, t, ((1,), (0,)))
        z = _dot(t, z, ((1,), (0,)))
    w_s = z * jax.lax.rsqrt(s)

    pw = _dot(p_s, w_s, ((1,), (0,)))

    @pl.when(k == 0)
    def _init():
        m_ref[...] = jnp.zeros_like(m_ref)

    q = _F // 4
    for j in range(4):
        mcq = _dot(pw, p_s[j * q:(j + 1) * q, :], ((1,), (1,)))
        m_ref[:, j * q:(j + 1) * q] += mcq


def _apply_kernel(x_ref, m_ref, mu_ref, o_ref):
    xc = x_ref[...] - mu_ref[...]
    o_ref[...] = jax.lax.dot_general(
        xc, m_ref[...], (((1,), (0,)), ((), ())),
        preferred_element_type=jnp.float32)


def kernel(x, shuffle_idx):
    n_rows, f = x.shape
    assert f == _F
    cf = shuffle_idx.astype(jnp.int32)
    cf3 = cf.reshape(_S, 1, _SW)

    blk = 1024
    blk_stats = 1024

    mean, gram = pl.pallas_call(
        functools.partial(_stats_kernel, inv_n=1.0 / n_rows,
                          last_k=n_rows // blk_stats - 1),
        grid=(n_rows // blk_stats,),
        in_specs=[pl.BlockSpec((blk_stats, _F), lambda k: (k, 0))],
        out_specs=[
            pl.BlockSpec((1, _F), lambda k: (0, 0)),
            pl.BlockSpec((_F, _F), lambda k: (0, 0)),
        ],
        out_shape=[
            jax.ShapeDtypeStruct((1, _F), jnp.float32),
            jax.ShapeDtypeStruct((_F, _F), jnp.float32),
        ],
        compiler_params=pltpu.CompilerParams(
            dimension_semantics=("arbitrary",)),
    )(x)

    m = pl.pallas_call(
        functools.partial(_whiten_kernel, n=float(n_rows)),
        grid=(_S,),
        in_specs=[
            pl.BlockSpec((_F, _F), lambda k: (0, 0)),
            pl.BlockSpec((1, 1, _SW), lambda k: (k, 0, 0)),
            pl.BlockSpec((1, _F), lambda k: (0, 0)),
        ],
        out_specs=pl.BlockSpec((_F, _F), lambda k: (0, 0)),
        out_shape=jax.ShapeDtypeStruct((_F, _F), jnp.float32),
        compiler_params=pltpu.CompilerParams(
            dimension_semantics=("arbitrary",)),
    )(gram, cf3, mean)

    out = pl.pallas_call(
        _apply_kernel,
        grid=(n_rows // blk,),
        in_specs=[
            pl.BlockSpec((blk, _F), lambda k: (k, 0)),
            pl.BlockSpec((_F, _F), lambda k: (0, 0)),
            pl.BlockSpec((1, _F), lambda k: (0, 0)),
        ],
        out_specs=pl.BlockSpec((blk, _F), lambda k: (k, 0)),
        out_shape=jax.ShapeDtypeStruct((n_rows, _F), jnp.float32),
        compiler_params=pltpu.CompilerParams(
            dimension_semantics=("arbitrary",)),
    )(x, m, mean)

    return out


# speedup vs baseline: 4.4298x; 1.1814x over previous
import functools

import jax
import jax.numpy as jnp
from jax.experimental import pallas as pl
from jax.experimental.pallas import tpu as pltpu

_F = 2048
_G = 32
_D = 64
_PACK = 4
_S = _G // _PACK
_SW = _PACK * _D
_NS_ITERS = 12


def _stats_kernel(x_ref, mean_ref, gram_ref, *, inv_n, last_k):
    k = pl.program_id(0)

    @pl.when(k == 0)
    def _init():
        mean_ref[...] = jnp.zeros_like(mean_ref)
        gram_ref[...] = jnp.zeros_like(gram_ref)

    xb = x_ref[...]
    mean_ref[...] += jnp.sum(xb, axis=0, keepdims=True)
    for bi in range(_S):
        xi = xb[:, bi * _SW:(bi + 1) * _SW]
        for bj in range(bi, _S):
            xj = xb[:, bj * _SW:(bj + 1) * _SW]
            gram_ref[bi * _SW:(bi + 1) * _SW, bj * _SW:(bj + 1) * _SW] += (
                jax.lax.dot_general(xi, xj, (((0,), (0,)), ((), ())),
                                    preferred_element_type=jnp.float32))

    @pl.when(k == last_k)
    def _finish():
        mean_ref[...] *= inv_n


def _dot(a, b, dims):
    return jax.lax.dot_general(a, b, (dims, ((), ())),
                               preferred_element_type=jnp.float32)


def _whiten_kernel(g_ref, cf_ref, mu_ref, m_ref, *, n):
    k = pl.program_id(0)

    ri = jax.lax.broadcasted_iota(jnp.int32, (_F, _SW), 0)
    p_s = (ri == cf_ref[0]).astype(jnp.float32)

    gp = _dot(g_ref[...], p_s, ((1,), (0,)))
    gu = _dot(p_s, gp, ((0,), (0,)))
    gd = jnp.zeros((_SW, _SW), jnp.float32)
    for b in range(_S):
        pb = p_s[b * _SW:(b + 1) * _SW, :]
        db = g_ref[b * _SW:(b + 1) * _SW, b * _SW:(b + 1) * _SW]
        gd = gd + _dot(pb, _dot(db, pb, ((1,), (0,))), ((0,), (0,)))
    gs = gu + gu.T - gd
    mu_s = _dot(mu_ref[...], p_s, ((1,), (0,)))
    outer = _dot(mu_s, mu_s, ((0,), (0,)))

    ri2 = jax.lax.broadcasted_iota(jnp.int32, (_SW, _SW), 0)
    ci2 = jax.lax.broadcasted_iota(jnp.int32, (_SW, _SW), 1)
    mask = ((ri2 // _D) == (ci2 // _D)).astype(jnp.float32)
    eye = (ri2 == ci2).astype(jnp.float32)

    cov = (gs - n * outer) * mask * (1.0 / _G)
    rowsum = jnp.sum(jnp.abs(cov), axis=-1, keepdims=True)
    s = jnp.maximum(jnp.max(rowsum), 1e-30)
    a = cov * (1.0 / s)

    y = a
    z = eye
    for _ in range(_NS_ITERS):
        zy = _dot(z, y, ((1,), (0,)))
        t = 1.5 * eye - 0.5 * zy
        y = _dot(y, t, ((1,), (0,)))
        z = _dot(t, z, ((1,), (0,)))
    w_s = z * jax.lax.rsqrt(s)

    pw = _dot(p_s, w_s, ((1,), (0,)))

    @pl.when(k == 0)
    def _init():
        m_ref[...] = jnp.zeros_like(m_ref)

    q = _F // 4
    for j in range(4):
        mcq = _dot(pw, p_s[j * q:(j + 1) * q, :], ((1,), (1,)))
        m_ref[:, j * q:(j + 1) * q] += mcq


def _apply_kernel(x_ref, m_ref, mu_ref, o_ref):
    xc = x_ref[...] - mu_ref[...]
    o_ref[...] = jax.lax.dot_general(
        xc, m_ref[...], (((1,), (0,)), ((), ())),
        preferred_element_type=jnp.float32)


def kernel(x, shuffle_idx):
    n_rows, f = x.shape
    assert f == _F
    cf = shuffle_idx.astype(jnp.int32)
    cf3 = cf.reshape(_S, 1, _SW)

    blk = 1024
    blk_stats = 1024

    mean, gram = pl.pallas_call(
        functools.partial(_stats_kernel, inv_n=1.0 / n_rows,
                          last_k=n_rows // blk_stats - 1),
        grid=(n_rows // blk_stats,),
        in_specs=[pl.BlockSpec((blk_stats, _F), lambda k: (k, 0))],
        out_specs=[
            pl.BlockSpec((1, _F), lambda k: (0, 0)),
            pl.BlockSpec((_F, _F), lambda k: (0, 0)),
        ],
        out_shape=[
            jax.ShapeDtypeStruct((1, _F), jnp.float32),
            jax.ShapeDtypeStruct((_F, _F), jnp.float32),
        ],
        compiler_params=pltpu.CompilerParams(
            dimension_semantics=("arbitrary",)),
    )(x)

    m = pl.pallas_call(
        functools.partial(_whiten_kernel, n=float(n_rows)),
        grid=(_S,),
        in_specs=[
            pl.BlockSpec((_F, _F), lambda k: (0, 0)),
            pl.BlockSpec((1, 1, _SW), lambda k: (k, 0, 0)),
            pl.BlockSpec((1, _F), lambda k: (0, 0)),
        ],
        out_specs=pl.BlockSpec((_F, _F), lambda k: (0, 0)),
        out_shape=jax.ShapeDtypeStruct((_F, _F), jnp.float32),
        compiler_params=pltpu.CompilerParams(
            dimension_semantics=("arbitrary",)),
    )(gram, cf3, mean)

    out = pl.pallas_call(
        _apply_kernel,
        grid=(n_rows // blk,),
        in_specs=[
            pl.BlockSpec((blk, _F), lambda k: (k, 0)),
            pl.BlockSpec((_F, _F), lambda k: (0, 0)),
            pl.BlockSpec((1, _F), lambda k: (0, 0)),
        ],
        out_specs=pl.BlockSpec((blk, _F), lambda k: (k, 0)),
        out_shape=jax.ShapeDtypeStruct((n_rows, _F), jnp.float32),
        compiler_params=pltpu.CompilerParams(
            dimension_semantics=("arbitrary",)),
    )(x, m, mean)

    return out
